# baseline (speedup 1.0000x reference)
import functools

import jax
import jax.numpy as jnp
from jax import lax
from jax.experimental import pallas as pl
from jax.experimental.pallas import tpu as pltpu
from jax.experimental.pallas import tpu_sc as plsc

N_IN = 4096
N_OUT = 4096
NNZ = 16777
BATCH = 1024

LANES = 16
NUM_CORES = 2
NUM_SUBCORES = 16
NUM_WORKERS = NUM_CORES * NUM_SUBCORES

E_FULL = (NNZ // LANES) * LANES
E_TAIL = NNZ - E_FULL

ROWS_PER_WORKER = BATCH // NUM_WORKERS
R = 8
PASSES = ROWS_PER_WORKER // R


def _sc_body(x_hbm, src_hbm, dst_hbm, w_hbm, bias_hbm, out_hbm,
             xbuf, outbuf, srcv, dstv, wv, biasv):
    wid = lax.axis_index("s") * NUM_CORES + lax.axis_index("c")

    pltpu.sync_copy(src_hbm, srcv)
    pltpu.sync_copy(dst_hbm, dstv)
    pltpu.sync_copy(w_hbm, wv)
    pltpu.sync_copy(bias_hbm, biasv)

    tail_mask = lax.iota(jnp.int32, LANES) < E_TAIL

    for p in range(PASSES):
        base = (wid * ROWS_PER_WORKER + p * R) * N_IN
        pltpu.sync_copy(x_hbm.at[pl.ds(base, R * N_IN)], xbuf)

        @plsc.parallel_loop(0, N_OUT, step=LANES, unroll=4)
        def _init(off):
            off = pl.multiple_of(off, LANES)
            b16 = biasv[pl.ds(off, LANES)]
            for r in range(R):
                outbuf[pl.ds(off + r * N_OUT, LANES)] = b16

        @plsc.parallel_loop(0, E_FULL, step=LANES, unroll=8)
        def _edges(off):
            off = pl.multiple_of(off, LANES)
            s16 = srcv[pl.ds(off, LANES)]
            d16 = dstv[pl.ds(off, LANES)]
            w16 = wv[pl.ds(off, LANES)]
            for r in range(R):
                vals = plsc.load_gather(xbuf, [s16 + (r * N_IN)]) * w16
                plsc.addupdate_scatter(outbuf, [d16 + (r * N_OUT)], vals)

        s16 = srcv[pl.ds(E_FULL - LANES + E_TAIL, LANES)]
        d16 = dstv[pl.ds(E_FULL - LANES + E_TAIL, LANES)]
        w16 = wv[pl.ds(E_FULL - LANES + E_TAIL, LANES)]
        lane_ok = lax.iota(jnp.int32, LANES) >= (LANES - E_TAIL)
        for r in range(R):
            vals = plsc.load_gather(xbuf, [s16 + (r * N_IN)], mask=lane_ok)
            plsc.addupdate_scatter(
                outbuf, [d16 + (r * N_OUT)], vals * w16, mask=lane_ok)

        pltpu.sync_copy(outbuf, out_hbm.at[pl.ds(base, R * N_OUT)])


_sc_kernel = functools.partial(
    pl.kernel,
    out_type=jax.ShapeDtypeStruct((BATCH * N_OUT,), jnp.float32),
    mesh=plsc.VectorSubcoreMesh(
        core_axis_name="c", subcore_axis_name="s",
        num_cores=NUM_CORES, num_subcores=NUM_SUBCORES),
    compiler_params=pltpu.CompilerParams(needs_layout_passes=False),
    scratch_types=[
        pltpu.VMEM((R * N_IN,), jnp.float32),
        pltpu.VMEM((R * N_OUT,), jnp.float32),
        pltpu.VMEM((NNZ,), jnp.int32),
        pltpu.VMEM((NNZ,), jnp.int32),
        pltpu.VMEM((NNZ,), jnp.float32),
        pltpu.VMEM((N_OUT,), jnp.float32),
    ],
)(_sc_body)


@jax.jit
def kernel(x, w_flat, bias, src_idx, dst_idx):
    out = _sc_kernel(x.reshape(-1), src_idx.astype(jnp.int32),
                     dst_idx.astype(jnp.int32), w_flat, bias)
    return out.reshape(BATCH, N_OUT)


# speedup vs baseline: 1.0133x; 1.0133x over previous
import functools

import jax
import jax.numpy as jnp
from jax import lax
from jax.experimental import pallas as pl
from jax.experimental.pallas import tpu as pltpu
from jax.experimental.pallas import tpu_sc as plsc

N_IN = 4096
N_OUT = 4096
NNZ = 16777
BATCH = 1024

LANES = 16
NUM_CORES = 2
NUM_SUBCORES = 16
NUM_WORKERS = NUM_CORES * NUM_SUBCORES

E_PAD = ((NNZ + LANES - 1) // LANES) * LANES

ROWS_PER_WORKER = BATCH // NUM_WORKERS
R = 8
PASSES = ROWS_PER_WORKER // R

SRC_MASK = 4095


def _sc_body(x_hbm, edges_hbm, w_hbm, bias_hbm, out_hbm,
             xbuf, outbuf, edgev, wv, biasv):
    wid = lax.axis_index("s") * NUM_CORES + lax.axis_index("c")

    pltpu.sync_copy(edges_hbm, edgev)
    pltpu.sync_copy(w_hbm, wv)
    pltpu.sync_copy(bias_hbm, biasv)

    for p in range(PASSES):
        base = (wid * ROWS_PER_WORKER + p * R) * N_IN
        pltpu.sync_copy(x_hbm.at[pl.ds(base, R * N_IN)], xbuf)

        @plsc.parallel_loop(0, N_OUT, step=LANES, unroll=4)
        def _init(off):
            off = pl.multiple_of(off, LANES)
            b16 = biasv[pl.ds(off, LANES)]
            for r in range(R):
                outbuf[pl.ds(off + r * N_OUT, LANES)] = b16

        @plsc.parallel_loop(0, E_PAD, step=LANES, unroll=8)
        def _edges(off):
            off = pl.multiple_of(off, LANES)
            e16 = edgev[pl.ds(off, LANES)]
            w16 = wv[pl.ds(off, LANES)]
            s16 = e16 & SRC_MASK
            d16 = lax.shift_right_logical(e16, 12)
            for r in range(R):
                vals = plsc.load_gather(xbuf, [s16 + (r * N_IN)]) * w16
                plsc.addupdate_scatter(outbuf, [d16 + (r * N_OUT)], vals)

        pltpu.sync_copy(outbuf, out_hbm.at[pl.ds(base, R * N_OUT)])


_sc_kernel = functools.partial(
    pl.kernel,
    out_type=jax.ShapeDtypeStruct((BATCH * N_OUT,), jnp.float32),
    mesh=plsc.VectorSubcoreMesh(
        core_axis_name="c", subcore_axis_name="s",
        num_cores=NUM_CORES, num_subcores=NUM_SUBCORES),
    compiler_params=pltpu.CompilerParams(needs_layout_passes=False),
    scratch_types=[
        pltpu.VMEM((R * N_IN,), jnp.float32),
        pltpu.VMEM((R * N_OUT,), jnp.float32),
        pltpu.VMEM((E_PAD,), jnp.int32),
        pltpu.VMEM((E_PAD,), jnp.float32),
        pltpu.VMEM((N_OUT,), jnp.float32),
    ],
)(_sc_body)


@jax.jit
def kernel(x, w_flat, bias, src_idx, dst_idx):
    pad = E_PAD - NNZ
    src = src_idx.astype(jnp.int32)
    dst = dst_idx.astype(jnp.int32)
    packed = jnp.concatenate(
        [(dst << 12) | src, jnp.zeros((pad,), jnp.int32)])
    w = jnp.concatenate(
        [w_flat.astype(jnp.float32), jnp.zeros((pad,), jnp.float32)])
    out = _sc_kernel(x.reshape(-1), packed, w, bias)
    return out.reshape(BATCH, N_OUT)


# speedup vs baseline: 1.2308x; 1.2146x over previous
import functools

import jax
import jax.numpy as jnp
from jax import lax
from jax.experimental import pallas as pl
from jax.experimental.pallas import tpu as pltpu
from jax.experimental.pallas import tpu_sc as plsc

N_IN = 4096
N_OUT = 4096
NNZ = 16777
BATCH = 1024

LANES = 16
NUM_CORES = 2
NUM_SUBCORES = 16
NUM_WORKERS = NUM_CORES * NUM_SUBCORES

E_PAD = ((NNZ + LANES - 1) // LANES) * LANES

ROWS_PER_WORKER = BATCH // NUM_WORKERS
R = 8
PASSES = ROWS_PER_WORKER // R

SRC_MASK = 4095


def _sc_body(x_hbm, edges_hbm, w_hbm, bias_hbm, out_hbm,
             xbuf, outbuf, edgev, wv, biasv):
    wid = lax.axis_index("s") * NUM_CORES + lax.axis_index("c")

    pltpu.sync_copy(edges_hbm, edgev)
    pltpu.sync_copy(w_hbm, wv)
    pltpu.sync_copy(bias_hbm, biasv)

    for p in range(PASSES):
        base = (wid * ROWS_PER_WORKER + p * R) * N_IN
        pltpu.sync_copy(x_hbm.at[pl.ds(base, R * N_IN)], xbuf)

        @plsc.parallel_loop(0, N_OUT, step=LANES, unroll=4)
        def _init(off):
            off = pl.multiple_of(off, LANES)
            b16 = biasv[pl.ds(off, LANES)]
            for r in range(R):
                outbuf[pl.ds(off + r * N_OUT, LANES)] = b16

        @plsc.parallel_loop(0, E_PAD, step=LANES, unroll=2)
        def _edges(off):
            off = pl.multiple_of(off, LANES)
            e16 = edgev[pl.ds(off, LANES)]
            w16 = wv[pl.ds(off, LANES)]
            s16 = e16 & SRC_MASK
            d16 = lax.shift_right_logical(e16, 12)
            for r in range(R):
                vals = plsc.load_gather(xbuf, [s16 + (r * N_IN)]) * w16
                plsc.addupdate_scatter(outbuf, [d16 + (r * N_OUT)], vals)

        pltpu.sync_copy(outbuf, out_hbm.at[pl.ds(base, R * N_OUT)])


_sc_kernel = functools.partial(
    pl.kernel,
    out_type=jax.ShapeDtypeStruct((BATCH * N_OUT,), jnp.float32),
    mesh=plsc.VectorSubcoreMesh(
        core_axis_name="c", subcore_axis_name="s",
        num_cores=NUM_CORES, num_subcores=NUM_SUBCORES),
    compiler_params=pltpu.CompilerParams(needs_layout_passes=False),
    scratch_types=[
        pltpu.VMEM((R * N_IN,), jnp.float32),
        pltpu.VMEM((R * N_OUT,), jnp.float32),
        pltpu.VMEM((E_PAD,), jnp.int32),
        pltpu.VMEM((E_PAD,), jnp.float32),
        pltpu.VMEM((N_OUT,), jnp.float32),
    ],
)(_sc_body)


@jax.jit
def kernel(x, w_flat, bias, src_idx, dst_idx):
    pad = E_PAD - NNZ
    src = src_idx.astype(jnp.int32)
    dst = dst_idx.astype(jnp.int32)
    packed = jnp.concatenate(
        [(dst << 12) | src, jnp.zeros((pad,), jnp.int32)])
    w = jnp.concatenate(
        [w_flat.astype(jnp.float32), jnp.zeros((pad,), jnp.float32)])
    out = _sc_kernel(x.reshape(-1), packed, w, bias)
    return out.reshape(BATCH, N_OUT)


# speedup vs baseline: 1.8365x; 1.4921x over previous
import functools

import jax
import jax.numpy as jnp
from jax import lax
from jax.experimental import pallas as pl
from jax.experimental.pallas import tpu as pltpu
from jax.experimental.pallas import tpu_sc as plsc

N_IN = 4096
N_OUT = 4096
NNZ = 16777
BATCH = 1024

LANES = 16
NUM_CORES = 2
NUM_SUBCORES = 16
NUM_WORKERS = NUM_CORES * NUM_SUBCORES

E_PAD = ((NNZ + LANES - 1) // LANES) * LANES

ROWS_PER_WORKER = BATCH // NUM_WORKERS
R = 8
PASSES = ROWS_PER_WORKER // R

SRC_MASK = 4095


def _sc_body(x_hbm, ew_hbm, bias_hbm, out_hbm, xbuf, outbuf, ewv, biasv):
    wid = lax.axis_index("s") * NUM_CORES + lax.axis_index("c")

    pltpu.sync_copy(ew_hbm, ewv)
    pltpu.sync_copy(bias_hbm, biasv)

    for p in range(PASSES):
        base = (wid * ROWS_PER_WORKER + p * R) * N_IN
        pltpu.sync_copy(x_hbm.at[pl.ds(base, R * N_IN)], xbuf)

        @plsc.parallel_loop(0, N_OUT, step=LANES, unroll=4)
        def _init(off):
            off = pl.multiple_of(off, LANES)
            b16 = biasv[pl.ds(off, LANES)]
            for r in range(R):
                outbuf[pl.ds(off + r * N_OUT, LANES)] = b16

        @plsc.parallel_loop(0, E_PAD, step=LANES, unroll=2)
        def _edges(off):
            off = pl.multiple_of(off, LANES)
            e16 = ewv[pl.ds(off, LANES)]
            w16 = plsc.bitcast(ewv[pl.ds(off + E_PAD, LANES)], jnp.float32)
            s16 = e16 & SRC_MASK
            d16 = lax.shift_right_logical(e16, 12)
            for r in range(R):
                vals = plsc.load_gather(xbuf, [s16 + (r * N_IN)]) * w16
                plsc.addupdate_scatter(outbuf, [d16 + (r * N_OUT)], vals)

        pltpu.sync_copy(outbuf, out_hbm.at[pl.ds(base, R * N_OUT)])


_sc_kernel = functools.partial(
    pl.kernel,
    out_type=jax.ShapeDtypeStruct((BATCH * N_OUT,), jnp.float32),
    mesh=plsc.VectorSubcoreMesh(
        core_axis_name="c", subcore_axis_name="s",
        num_cores=NUM_CORES, num_subcores=NUM_SUBCORES),
    compiler_params=pltpu.CompilerParams(needs_layout_passes=False),
    scratch_types=[
        pltpu.VMEM((R * N_IN,), jnp.float32),
        pltpu.VMEM((R * N_OUT,), jnp.float32),
        pltpu.VMEM((2 * E_PAD,), jnp.int32),
        pltpu.VMEM((N_OUT,), jnp.float32),
    ],
)(_sc_body)


def _interleave(a):
    return a.reshape(LANES, E_PAD // LANES).T.reshape(-1)


@jax.jit
def kernel(x, w_flat, bias, src_idx, dst_idx):
    pad = E_PAD - NNZ
    src = src_idx.astype(jnp.int32)
    dst = dst_idx.astype(jnp.int32)
    packed = jnp.concatenate(
        [(dst << 12) | src, jnp.zeros((pad,), jnp.int32)])
    wbits = jnp.concatenate(
        [lax.bitcast_convert_type(w_flat, jnp.int32),
         jnp.zeros((pad,), jnp.int32)])
    ew = jnp.concatenate([_interleave(packed), _interleave(wbits)])
    out = _sc_kernel(x.reshape(-1), ew, bias)
    return out.reshape(BATCH, N_OUT)
